# baseline (speedup 1.0000x reference)
import functools
import jax
import jax.numpy as jnp
from jax import lax
from jax.experimental import pallas as pl
from jax.experimental.pallas import tpu as pltpu
from jax.experimental.pallas import tpu_sc as plsc

def _make(dtype):
    mesh = plsc.VectorSubcoreMesh(core_axis_name="c", subcore_axis_name="s")
    @functools.partial(
        pl.kernel,
        out_type=jax.ShapeDtypeStruct((16, 64), dtype),
        mesh=mesh,
        scratch_types=[pltpu.VMEM((16, 64), dtype), pltpu.SemaphoreType.DMA],
    )
    def k(table_hbm, out_hbm, buf, sem):
        wid = lax.axis_index("s") * 2 + lax.axis_index("c")

        @pl.when(wid == 0)
        def _():
            pltpu.sync_copy(table_hbm.at[pl.ds(0, 16)], buf)
            pltpu.sync_copy(buf, out_hbm)
    return k

def kernel(x, grid_embedding):
    return _make(grid_embedding.dtype)(grid_embedding)
